# Initial kernel scaffold; baseline (speedup 1.0000x reference)
import jax
import jax.numpy as jnp
from jax.experimental import pallas as pl


def kernel(x):
    raise NotImplementedError("write your pallas kernel here")


# speedup vs baseline: 1.7632x; 1.7632x over previous
import functools

import jax
import jax.numpy as jnp
from jax import lax
from jax.experimental import pallas as pl
from jax.experimental.pallas import tpu as pltpu
from jax.experimental.pallas import tpu_sc as plsc

_L = 2048
_B = 4
_LIMIT = 256
_NROWS = _B * _L
_NW = 32
_RPW = _NROWS // _NW
_WIN = _LIMIT + 16
_BUF = 544
_AMAX = _L - _WIN
_NBUF = 4


def _row_scalars(g):
    i = lax.rem(g, _L)
    a = jnp.minimum(i - lax.rem(i, 16), _AMAX)
    src = pl.multiple_of(g * _L + a, 16)
    return i, i - a, src


def _emit_row(win, obuf, t, i, r, lane):
    colbase = i + lane
    obase = t * _LIMIT
    for k in range(_LIMIT // 16):
        v = win[pl.ds(r + 16 * k, 16)]
        v = jnp.where(colbase < (_L - 16 * k), v, 0.0)
        obuf[pl.ds(obase + 16 * k, 16)] = v


def _sc_body(x_ref, out_ref, w0, w1, w2, w3, obuf, s0, s1, s2, s3):
    wins = (w0, w1, w2, w3)
    sems = (s0, s1, s2, s3)
    wid = lax.axis_index("s") * 2 + lax.axis_index("c")
    g0 = wid * _RPW
    lane = lax.iota(jnp.int32, 16)

    def _issue(t_local, p):
        g = g0 + jnp.minimum(t_local, _RPW - 1)
        _, _, src = _row_scalars(g)
        pltpu.async_copy(
            x_ref.at[pl.ds(src, _WIN)], wins[p].at[pl.ds(0, _WIN)], sems[p]
        )

    def _wait(p):
        pltpu.make_async_copy(
            x_ref.at[pl.ds(0, _WIN)], wins[p].at[pl.ds(0, _WIN)], sems[p]
        ).wait()

    for p in range(_NBUF):
        _issue(p, p)

    def _step(it, carry):
        for p in range(_NBUF):
            t = it * _NBUF + p
            _wait(p)
            i, r, _ = _row_scalars(g0 + t)
            _emit_row(wins[p], obuf, t, i, r, lane)
            _issue(t + _NBUF, p)
        return carry

    lax.fori_loop(0, _RPW // _NBUF, _step, 0, unroll=False)

    for p in range(_NBUF):
        _wait(p)

    pltpu.sync_copy(obuf, out_ref.at[pl.ds(g0 * _LIMIT, _RPW * _LIMIT)])


@jax.jit
def _run(xflat):
    call = pl.kernel(
        _sc_body,
        out_type=jax.ShapeDtypeStruct((_NROWS * _LIMIT,), jnp.float32),
        mesh=plsc.VectorSubcoreMesh(core_axis_name="c", subcore_axis_name="s"),
        scratch_types=[
            pltpu.VMEM((_BUF,), jnp.float32),
            pltpu.VMEM((_BUF,), jnp.float32),
            pltpu.VMEM((_BUF,), jnp.float32),
            pltpu.VMEM((_BUF,), jnp.float32),
            pltpu.VMEM((_RPW * _LIMIT,), jnp.float32),
            pltpu.SemaphoreType.DMA,
            pltpu.SemaphoreType.DMA,
            pltpu.SemaphoreType.DMA,
            pltpu.SemaphoreType.DMA,
        ],
    )
    return call(xflat)


def kernel(x):
    B, L, D = x.shape
    out = _run(x.reshape(-1))
    return out.reshape(B, L, _LIMIT)


# speedup vs baseline: 1.9565x; 1.1096x over previous
import functools

import jax
import jax.numpy as jnp
from jax import lax
from jax.experimental import pallas as pl
from jax.experimental.pallas import tpu as pltpu
from jax.experimental.pallas import tpu_sc as plsc

_L = 2048
_B = 4
_LIMIT = 256
_NROWS = _B * _L
_NW = 32
_RPW = _NROWS // _NW
_WIN = _LIMIT + 16
_BUF = 544
_AMAX = _L - _WIN
_NBUF = 16


def _row_scalars(g):
    i = lax.rem(g, _L)
    a = jnp.minimum(i - lax.rem(i, 16), _AMAX)
    src = pl.multiple_of(g * _L + a, 16)
    return i, i - a, src


def _emit_row(win, obuf, t, i, r, lane):
    colbase = i + lane
    obase = t * _LIMIT
    for k in range(_LIMIT // 16):
        v = win[pl.ds(r + 16 * k, 16)]
        v = jnp.where(colbase < (_L - 16 * k), v, 0.0)
        obuf[pl.ds(obase + 16 * k, 16)] = v


def _sc_body(x_ref, out_ref, *scr):
    wins = scr[:_NBUF]
    obuf = scr[_NBUF]
    sems = scr[_NBUF + 1:]
    wid = lax.axis_index("s") * 2 + lax.axis_index("c")
    g0 = wid * _RPW
    lane = lax.iota(jnp.int32, 16)

    def _issue(t_local, p):
        g = g0 + jnp.minimum(t_local, _RPW - 1)
        _, _, src = _row_scalars(g)
        pltpu.async_copy(
            x_ref.at[pl.ds(src, _WIN)], wins[p].at[pl.ds(0, _WIN)], sems[p]
        )

    def _wait(p):
        pltpu.make_async_copy(
            x_ref.at[pl.ds(0, _WIN)], wins[p].at[pl.ds(0, _WIN)], sems[p]
        ).wait()

    for p in range(_NBUF):
        _issue(p, p)

    def _step(it, carry):
        for p in range(_NBUF):
            t = it * _NBUF + p
            _wait(p)
            i, r, _ = _row_scalars(g0 + t)
            _emit_row(wins[p], obuf, t, i, r, lane)
            _issue(t + _NBUF, p)
        return carry

    lax.fori_loop(0, _RPW // _NBUF, _step, 0, unroll=False)

    for p in range(_NBUF):
        _wait(p)

    pltpu.sync_copy(obuf, out_ref.at[pl.ds(g0 * _LIMIT, _RPW * _LIMIT)])


@jax.jit
def _run(xflat):
    call = pl.kernel(
        _sc_body,
        out_type=jax.ShapeDtypeStruct((_NROWS * _LIMIT,), jnp.float32),
        mesh=plsc.VectorSubcoreMesh(core_axis_name="c", subcore_axis_name="s"),
        scratch_types=(
            [pltpu.VMEM((_BUF,), jnp.float32) for _ in range(_NBUF)]
            + [pltpu.VMEM((_RPW * _LIMIT,), jnp.float32)]
            + [pltpu.SemaphoreType.DMA for _ in range(_NBUF)]
        ),
    )
    return call(xflat)


def kernel(x):
    B, L, D = x.shape
    out = _run(x.reshape(-1))
    return out.reshape(B, L, _LIMIT)


# speedup vs baseline: 2.0547x; 1.0502x over previous
import functools

import jax
import jax.numpy as jnp
from jax import lax
from jax.experimental import pallas as pl
from jax.experimental.pallas import tpu as pltpu
from jax.experimental.pallas import tpu_sc as plsc

_L = 2048
_B = 4
_LIMIT = 256
_NROWS = _B * _L
_NW = 32
_GR = 16
_NG = _NROWS // _GR
_GPW = _NG // _NW
_WIN = _LIMIT + 2 * _GR
_BUF = 544
_AMAX = _L - _WIN
_NBUF = 2


def _emit_fast(win, obuf, orow):
    for t in range(_GR):
        for k in range(_LIMIT // 16):
            obuf[orow + t, pl.ds(16 * k, 16)] = win[t, pl.ds(t + 16 * k, 16)]


def _emit_masked(win, obuf, orow, ig, lane):
    rbase = ig - _AMAX
    lcol = lane + ig
    for t in range(_GR):
        for k in range(_LIMIT // 16):
            v = win[t, pl.ds(rbase + (t + 16 * k), 16)]
            v = jnp.where(lcol < (_L - t - 16 * k), v, 0.0)
            obuf[orow + t, pl.ds(16 * k, 16)] = v


def _sc_body(x_ref, out_ref, *scr):
    wins = scr[:_NBUF]
    obuf = scr[_NBUF]
    sems = scr[_NBUF + 1:_NBUF + 1 + _NBUF]
    osem = scr[_NBUF + 1 + _NBUF]
    wid = lax.axis_index("s") * 2 + lax.axis_index("c")
    lane = lax.iota(jnp.int32, 16)

    def _grp(j):
        g = wid + _NW * j
        G = g * _GR
        ig = lax.rem(G, _L)
        a = pl.multiple_of(jnp.minimum(ig, _AMAX), 16)
        return G, ig, a

    def _issue(j, p):
        G, _, a = _grp(jnp.minimum(j, _GPW - 1))
        pltpu.async_copy(
            x_ref.at[pl.ds(G, _GR), pl.ds(a, _WIN)],
            wins[p].at[:, pl.ds(0, _WIN)],
            sems[p],
        )

    def _wait(p):
        pltpu.make_async_copy(
            x_ref.at[pl.ds(0, _GR), pl.ds(0, _WIN)],
            wins[p].at[:, pl.ds(0, _WIN)],
            sems[p],
        ).wait()

    def _out_copy(j):
        G = (wid + _NW * j) * _GR
        return pltpu.make_async_copy(
            obuf.at[pl.ds(j * _GR, _GR), :],
            out_ref.at[pl.ds(G, _GR), :],
            osem,
        )

    for p in range(_NBUF):
        _issue(jnp.int32(p), p)

    def _step(it, carry):
        for p in range(_NBUF):
            j = it * _NBUF + p
            _wait(p)
            _, ig, _ = _grp(j)
            orow = j * _GR

            @pl.when(ig <= _AMAX)
            def _():
                _emit_fast(wins[p], obuf, orow)

            @pl.when(ig > _AMAX)
            def _():
                _emit_masked(wins[p], obuf, orow, ig, lane)

            _issue(j + _NBUF, p)
            _out_copy(j).start()
        return carry

    lax.fori_loop(0, _GPW // _NBUF, _step, 0, unroll=False)

    for p in range(_NBUF):
        _wait(p)
    for j in range(_GPW):
        _out_copy(jnp.int32(j)).wait()


@jax.jit
def _run(x2d):
    call = pl.kernel(
        _sc_body,
        out_type=jax.ShapeDtypeStruct((_NROWS, _LIMIT), jnp.float32),
        mesh=plsc.VectorSubcoreMesh(core_axis_name="c", subcore_axis_name="s"),
        compiler_params=pltpu.CompilerParams(use_tc_tiling_on_sc=False),
        scratch_types=(
            [pltpu.VMEM((_GR, _BUF), jnp.float32) for _ in range(_NBUF)]
            + [pltpu.VMEM((_GPW * _GR, _LIMIT), jnp.float32)]
            + [pltpu.SemaphoreType.DMA for _ in range(_NBUF)]
            + [pltpu.SemaphoreType.DMA]
        ),
    )
    return call(x2d)


def kernel(x):
    B, L, D = x.shape
    out = _run(x.reshape(B * L, D))
    return out.reshape(B, L, _LIMIT)


# speedup vs baseline: 3.3617x; 1.6361x over previous
import functools

import jax
import jax.numpy as jnp
from jax import lax
from jax.experimental import pallas as pl
from jax.experimental.pallas import tpu as pltpu
from jax.experimental.pallas import tpu_sc as plsc

_L = 2048
_B = 4
_LIMIT = 256
_NROWS = _B * _L
_NW = 32
_GR = 16
_NG = _NROWS // _GR
_GPW = _NG // _NW
_WIN = _LIMIT + 128
_BUF = 640
_AMAX = _L - _WIN
_NBUF = 2


def _row_chunks(win, t, rbase, lane):
    rot = lax.rem(lane + t, 16)
    seam = lane < (16 - t)
    c = win[t, pl.ds(rbase, 16)]
    for k in range(_LIMIT // 16):
        cn = win[t, pl.ds(rbase + 16 * (k + 1), 16)]
        if t == 0:
            v = c
        else:
            v = jnp.where(
                seam,
                jnp.take(c, rot, mode="wrap"),
                jnp.take(cn, rot, mode="wrap"),
            )
        yield k, v
        c = cn


def _emit_plain(win, obuf, orow, rbase, lane):
    for t in range(_GR):
        for k, v in _row_chunks(win, t, rbase, lane):
            obuf[orow + t, pl.ds(16 * k, 16)] = v


def _emit_masked(win, obuf, orow, ig, rbase, lane):
    lcol = lane + ig
    for t in range(_GR):
        for k, v in _row_chunks(win, t, rbase, lane):
            v = jnp.where(lcol < (_L - t - 16 * k), v, 0.0)
            obuf[orow + t, pl.ds(16 * k, 16)] = v


def _sc_body(x_ref, out_ref, *scr):
    wins = scr[:_NBUF]
    obuf = scr[_NBUF]
    sems = scr[_NBUF + 1:_NBUF + 1 + _NBUF]
    osem = scr[_NBUF + 1 + _NBUF]
    wid = lax.axis_index("s") * 2 + lax.axis_index("c")
    lane = lax.iota(jnp.int32, 16)

    def _grp(j):
        g = wid + _NW * j
        G = pl.multiple_of(g * _GR, _GR)
        ig = lax.rem(G, _L)
        a = pl.multiple_of(jnp.minimum(ig - lax.rem(ig, 128), _AMAX), 128)
        return G, ig, a

    def _issue(j, p):
        G, _, a = _grp(jnp.minimum(j, _GPW - 1))
        pltpu.async_copy(
            x_ref.at[pl.ds(G, _GR), pl.ds(a, _WIN)],
            wins[p].at[:, pl.ds(0, _WIN)],
            sems[p],
        )

    def _wait(p):
        pltpu.make_async_copy(
            x_ref.at[pl.ds(0, _GR), pl.ds(0, _WIN)],
            wins[p].at[:, pl.ds(0, _WIN)],
            sems[p],
        ).wait()

    def _out_copy(j):
        G = pl.multiple_of((wid + _NW * j) * _GR, _GR)
        return pltpu.make_async_copy(
            obuf.at[pl.ds(j * _GR, _GR), :],
            out_ref.at[pl.ds(G, _GR), :],
            osem,
        )

    for p in range(_NBUF):
        _issue(jnp.int32(p), p)

    def _step(it, carry):
        for p in range(_NBUF):
            j = it * _NBUF + p
            _wait(p)
            _, ig, a = _grp(j)
            rbase = pl.multiple_of(ig - a, 16)
            orow = j * _GR

            @pl.when(ig <= _AMAX + 112)
            def _():
                _emit_plain(wins[p], obuf, orow, rbase, lane)

            @pl.when(ig > _AMAX + 112)
            def _():
                _emit_masked(wins[p], obuf, orow, ig, rbase, lane)

            _issue(j + _NBUF, p)
            _out_copy(j).start()
        return carry

    lax.fori_loop(0, _GPW // _NBUF, _step, 0, unroll=False)

    for p in range(_NBUF):
        _wait(p)
    for j in range(_GPW):
        _out_copy(jnp.int32(j)).wait()


@jax.jit
def _run(x2d):
    call = pl.kernel(
        _sc_body,
        out_type=jax.ShapeDtypeStruct((_NROWS, _LIMIT), jnp.float32),
        mesh=plsc.VectorSubcoreMesh(core_axis_name="c", subcore_axis_name="s"),
        scratch_types=(
            [pltpu.VMEM((_GR, _BUF), jnp.float32) for _ in range(_NBUF)]
            + [pltpu.VMEM((_GPW * _GR, _LIMIT), jnp.float32)]
            + [pltpu.SemaphoreType.DMA for _ in range(_NBUF)]
            + [pltpu.SemaphoreType.DMA]
        ),
    )
    return call(x2d)


def kernel(x):
    B, L, D = x.shape
    out = _run(x.reshape(B * L, D))
    return out.reshape(B, L, _LIMIT)
